# Initial kernel scaffold; baseline (speedup 1.0000x reference)
import jax
import jax.numpy as jnp
from jax.experimental import pallas as pl


def kernel(node_embs_src, node_embs_dst, edge_index, W1, b1, W2, b2):
    raise NotImplementedError("write your pallas kernel here")


# speedup vs baseline: 4.0795x; 4.0795x over previous
import functools

import jax
import jax.numpy as jnp
from jax import lax
from jax.experimental import pallas as pl
from jax.experimental.pallas import tpu as pltpu
from jax.experimental.pallas import tpu_sc as plsc

_NC = 2
_NS = 16
_NW = _NC * _NS
_LANES = 16


def _proj_body(src_ref, dst_ref, w1a_ref, w1b_ref, b1_ref, ps_ref, pd_ref):
    ps_ref[...] = (
        jnp.dot(src_ref[...], w1a_ref[...], preferred_element_type=jnp.float32)
        + b1_ref[...]
    )
    pd_ref[...] = jnp.dot(
        dst_ref[...], w1b_ref[...], preferred_element_type=jnp.float32
    )


def _project(src, dst, w1a, w1b, b1):
    n, d = src.shape
    blk = 1000 if n % 1000 == 0 else n
    grid = n // blk
    return pl.pallas_call(
        _proj_body,
        grid=(grid,),
        in_specs=[
            pl.BlockSpec((blk, d), lambda i: (i, 0)),
            pl.BlockSpec((blk, d), lambda i: (i, 0)),
            pl.BlockSpec((d, d), lambda i: (0, 0)),
            pl.BlockSpec((d, d), lambda i: (0, 0)),
            pl.BlockSpec((1, d), lambda i: (0, 0)),
        ],
        out_specs=[
            pl.BlockSpec((blk, d), lambda i: (i, 0)),
            pl.BlockSpec((blk, d), lambda i: (i, 0)),
        ],
        out_shape=[
            jax.ShapeDtypeStruct((n, d), jnp.float32),
            jax.ShapeDtypeStruct((n, d), jnp.float32),
        ],
    )(src, dst, w1a, w1b, b1.reshape(1, d))


def _edge_scorer(n, d, e, chunk):
    epw = e // _NW
    nchunk = epw // chunk
    nk = d // _LANES

    mesh = plsc.VectorSubcoreMesh(core_axis_name="c", subcore_axis_name="s")

    def body(ps_hbm, pd_hbm, si_hbm, di_hbm, w2_hbm, b2_hbm, out_hbm,
             w2_v, b2_v, idx_s, idx_d, rows_s, rows_d, out_v, sem0, sem1):
        wid = lax.axis_index("s") * _NC + lax.axis_index("c")
        base = pl.multiple_of(wid * epw, 8)

        pltpu.sync_copy(w2_hbm, w2_v)
        pltpu.sync_copy(b2_hbm, b2_v)
        w2r = [w2_v[pl.ds(_LANES * k, _LANES)] for k in range(nk)]
        b2r = b2_v[...]

        sems = (sem0, sem1)

        def fire(g, b):
            off = pl.multiple_of(base + g * chunk, 8)
            pltpu.sync_copy(si_hbm.at[pl.ds(off, chunk)], idx_s.at[b])
            pltpu.sync_copy(di_hbm.at[pl.ds(off, chunk)], idx_d.at[b])
            pltpu.async_copy(ps_hbm.at[idx_s.at[b]], rows_s.at[b], sems[b])
            pltpu.async_copy(pd_hbm.at[idx_d.at[b]], rows_d.at[b], sems[b])

        def wait(b):
            pltpu.make_async_copy(ps_hbm.at[idx_s.at[b]], rows_s.at[b], sems[b]).wait()
            pltpu.make_async_copy(pd_hbm.at[idx_d.at[b]], rows_d.at[b], sems[b]).wait()

        lane = lax.iota(jnp.int32, _LANES)
        rot = [(lane + sh) & (_LANES - 1) for sh in (8, 4, 2, 1)]

        def compute(g, b):
            rs = rows_s.at[b]
            rd = rows_d.at[b]

            @pl.loop(0, chunk // _LANES)
            def group(gi):
                res = jnp.zeros((_LANES,), jnp.float32)
                for l in range(_LANES):
                    i = gi * _LANES + l
                    acc = (
                        jnp.maximum(
                            rs[i, pl.ds(0, _LANES)] + rd[i, pl.ds(0, _LANES)], 0.0
                        )
                        * w2r[0]
                    )
                    for k in range(1, nk):
                        sl = pl.ds(_LANES * k, _LANES)
                        acc = acc + jnp.maximum(rs[i, sl] + rd[i, sl], 0.0) * w2r[k]
                    for r in rot:
                        acc = acc + acc[r]
                    res = jnp.where(lane == l, acc, res)
                v = res + b2r
                out_v[pl.ds(gi * _LANES, _LANES)] = 1.0 / (1.0 + jnp.exp(-v))

            off = pl.multiple_of(base + g * chunk, 8)
            pltpu.sync_copy(out_v, out_hbm.at[pl.ds(off, chunk)])

        fire(0, 0)

        @pl.loop(0, nchunk - 1, step=2)
        def step(g):
            fire(g + 1, 1)
            wait(0)
            compute(g, 0)
            fire(g + 2, 0)
            wait(1)
            compute(g + 1, 1)

        wait(0)
        compute(nchunk - 1, 0)

    return pl.kernel(
        body,
        out_type=jax.ShapeDtypeStruct((e,), jnp.float32),
        mesh=mesh,
        scratch_types=[
            pltpu.VMEM((d,), jnp.float32),
            pltpu.VMEM((_LANES,), jnp.float32),
            pltpu.VMEM((2, chunk), jnp.int32),
            pltpu.VMEM((2, chunk), jnp.int32),
            pltpu.VMEM((2, chunk, d), jnp.float32),
            pltpu.VMEM((2, chunk, d), jnp.float32),
            pltpu.VMEM((chunk,), jnp.float32),
            pltpu.SemaphoreType.DMA,
            pltpu.SemaphoreType.DMA,
        ],
    )


@jax.jit
def kernel(node_embs_src, node_embs_dst, edge_index, W1, b1, W2, b2):
    n, d = node_embs_src.shape
    e = edge_index.shape[1]
    assert e % _NW == 0
    epw = e // _NW
    chunk = 80
    assert epw % chunk == 0 and (epw // chunk) % 2 == 1

    w1a = W1[:d]
    w1b = W1[d:]
    ps, pd = _project(node_embs_src, node_embs_dst, w1a, w1b, b1)

    si = edge_index[0]
    di = edge_index[1]
    w2 = W2[:, 0]
    b2v = jnp.broadcast_to(b2, (_LANES,))

    out = _edge_scorer(n, d, e, chunk)(ps, pd, si, di, w2, b2v)
    return out.reshape(e, 1)


# speedup vs baseline: 4.8068x; 1.1783x over previous
import functools

import jax
import jax.numpy as jnp
from jax import lax
from jax.experimental import pallas as pl
from jax.experimental.pallas import tpu as pltpu
from jax.experimental.pallas import tpu_sc as plsc

_NC = 2
_NS = 16
_NW = _NC * _NS
_LANES = 16


def _proj_body(embs_ref, w_ref, b_ref, out_ref):
    out_ref[...] = (
        jnp.dot(embs_ref[...], w_ref[0], preferred_element_type=jnp.float32)
        + b_ref[0]
    )


def _project(embs2, wstk, bstk):
    n2, d = embs2.shape
    n = n2 // 2
    blk = 1000 if n % 1000 == 0 else n
    grid = n // blk
    return pl.pallas_call(
        _proj_body,
        grid=(2, grid),
        in_specs=[
            pl.BlockSpec((blk, d), lambda t, i: (t * grid + i, 0)),
            pl.BlockSpec((1, d, d), lambda t, i: (t, 0, 0)),
            pl.BlockSpec((1, 1, d), lambda t, i: (t, 0, 0)),
        ],
        out_specs=pl.BlockSpec((blk, d), lambda t, i: (t * grid + i, 0)),
        out_shape=jax.ShapeDtypeStruct((n2, d), jnp.float32),
    )(embs2, wstk, bstk)


def _edge_scorer(d, e, chunk):
    epw = e // _NW
    nchunk = epw // chunk
    nk = d // _LANES
    c2 = 2 * chunk

    mesh = plsc.VectorSubcoreMesh(core_axis_name="c", subcore_axis_name="s")

    def body(tbl_hbm, idx_hbm, w2_hbm, b2_hbm, out_hbm,
             w2_v, b2_v, idx_v, rows, out_v, sem0, sem1):
        wid = lax.axis_index("s") * _NC + lax.axis_index("c")
        base = pl.multiple_of(wid * epw, 8)

        pltpu.sync_copy(w2_hbm, w2_v)
        pltpu.sync_copy(b2_hbm, b2_v)
        pltpu.sync_copy(idx_hbm.at[pl.ds(pl.multiple_of(wid * 2 * epw, 8), 2 * epw)], idx_v)

        w2r = [w2_v[pl.ds(_LANES * k, _LANES)] for k in range(nk)]
        b2r = b2_v[...]
        sems = (sem0, sem1)

        lane = lax.iota(jnp.int32, _LANES)
        rot = [(lane + sh) & (_LANES - 1) for sh in (8, 4, 2, 1)]

        def gather(g, b):
            idx = idx_v.at[pl.ds(g * c2, c2)]
            return pltpu.make_async_copy(tbl_hbm.at[idx], rows.at[b], sems[b])

        def fire(g, b):
            idx = idx_v.at[pl.ds(g * c2, c2)]
            pltpu.async_copy(tbl_hbm.at[idx], rows.at[b], sems[b])

        def compute(g, b):
            rows_b = rows.at[b]

            @pl.loop(0, chunk // _LANES)
            def group(gi):
                res = jnp.zeros((_LANES,), jnp.float32)
                for l in range(_LANES):
                    i = gi * _LANES + l
                    acc = (
                        jnp.maximum(
                            rows_b[i, pl.ds(0, _LANES)]
                            + rows_b[chunk + i, pl.ds(0, _LANES)],
                            0.0,
                        )
                        * w2r[0]
                    )
                    for k in range(1, nk):
                        sl = pl.ds(_LANES * k, _LANES)
                        acc = (
                            acc
                            + jnp.maximum(rows_b[i, sl] + rows_b[chunk + i, sl], 0.0)
                            * w2r[k]
                        )
                    for r in rot:
                        acc = acc + acc[r]
                    res = jnp.where(lane == l, acc, res)
                v = res + b2r
                out_v[pl.ds(g * chunk + gi * _LANES, _LANES)] = 1.0 / (
                    1.0 + jnp.exp(-v)
                )

        fire(0, 0)

        @pl.loop(0, nchunk - 1, step=2)
        def step(g):
            fire(g + 1, 1)
            gather(g, 0).wait()
            compute(g, 0)
            fire(g + 2, 0)
            gather(g + 1, 1).wait()
            compute(g + 1, 1)

        gather(nchunk - 1, 0).wait()
        compute(nchunk - 1, 0)

        pltpu.sync_copy(out_v, out_hbm.at[pl.ds(base, epw)])

    return pl.kernel(
        body,
        out_type=jax.ShapeDtypeStruct((e,), jnp.float32),
        mesh=mesh,
        scratch_types=[
            pltpu.VMEM((d,), jnp.float32),
            pltpu.VMEM((_LANES,), jnp.float32),
            pltpu.VMEM((2 * epw,), jnp.int32),
            pltpu.VMEM((2, c2, d), jnp.float32),
            pltpu.VMEM((epw,), jnp.float32),
            pltpu.SemaphoreType.DMA,
            pltpu.SemaphoreType.DMA,
        ],
    )


@jax.jit
def kernel(node_embs_src, node_embs_dst, edge_index, W1, b1, W2, b2):
    n, d = node_embs_src.shape
    e = edge_index.shape[1]
    assert e % _NW == 0
    epw = e // _NW
    chunk = 80
    assert epw % chunk == 0 and (epw // chunk) % 2 == 1
    nchunk = epw // chunk

    embs2 = jnp.concatenate([node_embs_src, node_embs_dst], axis=0)
    wstk = jnp.stack([W1[:d], W1[d:]])
    bstk = jnp.stack([b1.reshape(1, d), jnp.zeros((1, d), jnp.float32)])
    tbl = _project(embs2, wstk, bstk)

    si = edge_index[0].reshape(_NW, nchunk, 1, chunk)
    di = edge_index[1].reshape(_NW, nchunk, 1, chunk) + n
    idx_all = jnp.concatenate([si, di], axis=2).reshape(-1)

    w2 = W2[:, 0]
    b2v = jnp.broadcast_to(b2, (_LANES,))

    out = _edge_scorer(d, e, chunk)(tbl, idx_all, w2, b2v)
    return out.reshape(e, 1)
